# baseline (speedup 1.0000x reference)
import functools

import jax
import jax.numpy as jnp
from jax import lax
from jax.experimental import pallas as pl
from jax.experimental.pallas import tpu as pltpu
from jax.experimental.pallas import tpu_sc as plsc

RAW_OFFS = [0, 4, 16, 28, 38, 44, 50, 52]
TAB_ROWS = 431
D = 128
NC, NS = 2, 16
NW = NC * NS
CH = 128


def _make_sc_call(n_pad):
    rows_w = n_pad // NW
    nchunk = rows_w // CH
    mesh = plsc.VectorSubcoreMesh(core_axis_name="c", subcore_axis_name="s")

    @functools.partial(
        pl.kernel,
        out_type=jax.ShapeDtypeStruct((n_pad * D,), jnp.float32),
        mesh=mesh,
        scratch_types=[
            pltpu.VMEM((TAB_ROWS * D,), jnp.float32),
            pltpu.VMEM((54 * D,), jnp.float32),
            pltpu.VMEM((9, CH), jnp.int32),
            pltpu.VMEM((9, CH), jnp.int32),
            pltpu.VMEM((CH * D,), jnp.float32),
            pltpu.VMEM((CH * D,), jnp.float32),
            pltpu.SemaphoreType.DMA,
            pltpu.SemaphoreType.DMA,
            pltpu.SemaphoreType.DMA,
            pltpu.SemaphoreType.DMA,
        ],
        compiler_params=pltpu.CompilerParams(needs_layout_passes=False),
    )
    def body(nf_t, W0, W1, W2, W3, W4, W5, W6, W7, W8, out,
             tab, raw, idx0, idx1, ob0, ob1, si0, si1, so0, so1):
        wid = lax.axis_index("s") * NC + lax.axis_index("c")
        pltpu.sync_copy(W0, tab.at[pl.ds(0, 119 * D)])
        for w, off in zip((W1, W2, W3, W4, W5, W6, W7, W8), RAW_OFFS):
            pltpu.sync_copy(w, raw.at[pl.ds(off * D, w.shape[0])])

        def combine_row(dst, srcs):
            for k in range(D // 16):
                v = raw[pl.ds(srcs[0] * D + k * 16, 16)]
                for s in srcs[1:]:
                    v = v + raw[pl.ds(s * D + k * 16, 16)]
                tab[pl.ds(dst * D + k * 16, 16)] = v

        def combine2(dst_off, an, ao, bn, bo):
            def abody(a, _):
                def bbody(b, _):
                    combine_row(dst_off + a * bn + b, [ao + a, bo + b])
                    return 0
                lax.fori_loop(0, bn, bbody, 0)
                return 0
            lax.fori_loop(0, an, abody, 0)

        combine2(119, 4, 0, 12, 4)
        combine2(167, 12, 16, 10, 28)

        def c56(a, _):
            def c6(b, _):
                for c in range(2):
                    for e in range(2):
                        combine_row(287 + ((a * 6 + b) * 2 + c) * 2 + e,
                                    [38 + a, 44 + b, 50 + c, 52 + e])
                return 0
            lax.fori_loop(0, 6, c6, 0)
            return 0
        lax.fori_loop(0, 6, c56, 0)

        idxs, obs = (idx0, idx1), (ob0, ob1)
        sis, sos = (si0, si1), (so0, so1)

        def in_slice(ci):
            return nf_t.at[:, pl.ds(wid * rows_w + ci * CH, CH)]

        def out_slice(ci):
            return out.at[pl.ds((wid * rows_w + ci * CH) * D, CH * D)]

        lane = lax.iota(jnp.int32, 16)

        def compute_chunk(idx_v, ob):
            def g_body(g, _):
                f = [idx_v[i, pl.ds(g * 16, 16)] for i in range(9)]
                rb = [
                    f[0] * D,
                    (119 + f[1] * 12 + f[2]) * D,
                    (167 + f[3] * 10 + f[4]) * D,
                    (287 + ((f[5] * 6 + f[6]) * 2 + f[7]) * 2 + f[8]) * D,
                ]
                obase = (lane + g * 16) * D

                def c_body(cb, _):
                    c0 = cb * 16
                    for u in range(16):
                        colv = (lane + (c0 + u)) & (D - 1)
                        acc = plsc.load_gather(tab, [rb[0] + colv])
                        acc = acc + plsc.load_gather(tab, [rb[1] + colv])
                        acc = acc + plsc.load_gather(tab, [rb[2] + colv])
                        acc = acc + plsc.load_gather(tab, [rb[3] + colv])
                        plsc.store_scatter(ob, [obase + colv], acc)
                    return 0

                lax.fori_loop(0, D // 16, c_body, 0)
                return 0
            lax.fori_loop(0, CH // 16, g_body, 0)

        pltpu.async_copy(in_slice(0), idx0, si0)
        pltpu.async_copy(in_slice(1), idx1, si1)

        def super_body(s, _):
            for b in range(2):
                ci = s * 2 + b
                pltpu.make_async_copy(in_slice(ci), idxs[b], sis[b]).wait()

                @pl.when(s > 0)
                def _():
                    pltpu.make_async_copy(obs[b], out_slice(ci), sos[b]).wait()

                compute_chunk(idxs[b], obs[b])
                pltpu.async_copy(obs[b], out_slice(ci), sos[b])

                @pl.when(ci + 2 < nchunk)
                def _():
                    pltpu.async_copy(in_slice(ci + 2), idxs[b], sis[b])
            return 0

        npair = nchunk // 2
        lax.fori_loop(0, npair, super_body, 0)
        if nchunk % 2:
            ci = nchunk - 1
            pltpu.make_async_copy(in_slice(ci), idxs[0], sis[0]).wait()
            pltpu.make_async_copy(obs[0], out_slice(ci), sos[0]).wait()
            compute_chunk(idxs[0], obs[0])
            pltpu.async_copy(obs[0], out_slice(ci), sos[0])
        for b in range(2):
            pltpu.make_async_copy(
                obs[b], out_slice(nchunk - 2 + b), sos[b]
            ).wait()

    return body


def kernel(node_features, W0, W1, W2, W3, W4, W5, W6, W7, W8):
    n = node_features.shape[0]
    n_pad = -(-n // (NW * CH)) * (NW * CH)
    nf_t = jnp.pad(node_features, ((0, n_pad - n), (0, 0))).T
    ws = [w.reshape(-1) for w in (W0, W1, W2, W3, W4, W5, W6, W7, W8)]
    out = _make_sc_call(n_pad)(nf_t, *ws)
    return out.reshape(n_pad, D)[:n]


# speedup vs baseline: 2.8985x; 2.8985x over previous
import functools

import jax
import jax.numpy as jnp
from jax import lax
from jax.experimental import pallas as pl
from jax.experimental.pallas import tpu as pltpu
from jax.experimental.pallas import tpu_sc as plsc

RAW_OFFS = [0, 4, 16, 28, 38, 44, 50, 52]
TAB_ROWS = 431
D = 128
NC, NS = 2, 16
NW = NC * NS
CH = 128


def _make_sc_call(n_pad):
    rows_w = n_pad // NW
    nchunk = rows_w // CH
    mesh = plsc.VectorSubcoreMesh(core_axis_name="c", subcore_axis_name="s")

    @functools.partial(
        pl.kernel,
        out_type=jax.ShapeDtypeStruct((n_pad * D,), jnp.float32),
        mesh=mesh,
        scratch_types=[
            pltpu.VMEM((TAB_ROWS * D,), jnp.float32),
            pltpu.VMEM((54 * D,), jnp.float32),
            pltpu.VMEM((9, CH), jnp.int32),
            pltpu.VMEM((9, CH), jnp.int32),
            pltpu.VMEM((CH * D,), jnp.float32),
            pltpu.VMEM((CH * D,), jnp.float32),
            pltpu.SemaphoreType.DMA,
            pltpu.SemaphoreType.DMA,
            pltpu.SemaphoreType.DMA,
            pltpu.SemaphoreType.DMA,
        ],
        compiler_params=pltpu.CompilerParams(needs_layout_passes=False),
    )
    def body(nf_t, W0, W1, W2, W3, W4, W5, W6, W7, W8, out,
             tab, raw, idx0, idx1, ob0, ob1, si0, si1, so0, so1):
        wid = lax.axis_index("s") * NC + lax.axis_index("c")
        pltpu.sync_copy(W0, tab.at[pl.ds(0, 119 * D)])
        for w, off in zip((W1, W2, W3, W4, W5, W6, W7, W8), RAW_OFFS):
            pltpu.sync_copy(w, raw.at[pl.ds(off * D, w.shape[0])])

        def combine_row(dst, srcs):
            for k in range(D // 16):
                v = raw[pl.ds(srcs[0] * D + k * 16, 16)]
                for s in srcs[1:]:
                    v = v + raw[pl.ds(s * D + k * 16, 16)]
                tab[pl.ds(dst * D + k * 16, 16)] = v

        def combine2(dst_off, an, ao, bn, bo):
            def abody(a, _):
                def bbody(b, _):
                    combine_row(dst_off + a * bn + b, [ao + a, bo + b])
                    return 0
                lax.fori_loop(0, bn, bbody, 0)
                return 0
            lax.fori_loop(0, an, abody, 0)

        combine2(119, 4, 0, 12, 4)
        combine2(167, 12, 16, 10, 28)

        def c56(a, _):
            def c6(b, _):
                for c in range(2):
                    for e in range(2):
                        combine_row(287 + ((a * 6 + b) * 2 + c) * 2 + e,
                                    [38 + a, 44 + b, 50 + c, 52 + e])
                return 0
            lax.fori_loop(0, 6, c6, 0)
            return 0
        lax.fori_loop(0, 6, c56, 0)

        idxs, obs = (idx0, idx1), (ob0, ob1)
        sis, sos = (si0, si1), (so0, so1)

        def in_slice(ci):
            return nf_t.at[:, pl.ds(wid * rows_w + ci * CH, CH)]

        def out_slice(ci):
            return out.at[pl.ds((wid * rows_w + ci * CH) * D, CH * D)]

        lane = lax.iota(jnp.int32, 16)

        def compute_chunk(idx_v, ob):
            def g_body(g, _):
                f = [idx_v[i, pl.ds(g * 16, 16)] for i in range(9)]
                rb = [
                    f[0] * D,
                    (119 + f[1] * 12 + f[2]) * D,
                    (167 + f[3] * 10 + f[4]) * D,
                    (287 + ((f[5] * 6 + f[6]) * 2 + f[7]) * 2 + f[8]) * D,
                ]
                obase = (lane + g * 16) * D

                @functools.partial(plsc.parallel_loop, 0, D, unroll=16)
                def c_body(c):
                    colv = (lane + c) & (D - 1)
                    acc = plsc.load_gather(tab, [rb[0] + colv])
                    acc = acc + plsc.load_gather(tab, [rb[1] + colv])
                    acc = acc + plsc.load_gather(tab, [rb[2] + colv])
                    acc = acc + plsc.load_gather(tab, [rb[3] + colv])
                    plsc.store_scatter(ob, [obase + colv], acc)

                return 0
            lax.fori_loop(0, CH // 16, g_body, 0)

        pltpu.async_copy(in_slice(0), idx0, si0)
        pltpu.async_copy(in_slice(1), idx1, si1)

        def super_body(s, _):
            for b in range(2):
                ci = s * 2 + b
                pltpu.make_async_copy(in_slice(ci), idxs[b], sis[b]).wait()

                @pl.when(s > 0)
                def _():
                    pltpu.make_async_copy(obs[b], out_slice(ci), sos[b]).wait()

                compute_chunk(idxs[b], obs[b])
                pltpu.async_copy(obs[b], out_slice(ci), sos[b])

                @pl.when(ci + 2 < nchunk)
                def _():
                    pltpu.async_copy(in_slice(ci + 2), idxs[b], sis[b])
            return 0

        npair = nchunk // 2
        lax.fori_loop(0, npair, super_body, 0)
        if nchunk % 2:
            ci = nchunk - 1
            pltpu.make_async_copy(in_slice(ci), idxs[0], sis[0]).wait()
            pltpu.make_async_copy(obs[0], out_slice(ci), sos[0]).wait()
            compute_chunk(idxs[0], obs[0])
            pltpu.async_copy(obs[0], out_slice(ci), sos[0])
        for b in range(2):
            pltpu.make_async_copy(
                obs[b], out_slice(nchunk - 2 + b), sos[b]
            ).wait()

    return body


def kernel(node_features, W0, W1, W2, W3, W4, W5, W6, W7, W8):
    n = node_features.shape[0]
    n_pad = -(-n // (NW * CH)) * (NW * CH)
    nf_t = jnp.pad(node_features, ((0, n_pad - n), (0, 0))).T
    ws = [w.reshape(-1) for w in (W0, W1, W2, W3, W4, W5, W6, W7, W8)]
    out = _make_sc_call(n_pad)(nf_t, *ws)
    return out.reshape(n_pad, D)[:n]
